# baseline (speedup 1.0000x reference)
import functools

import jax
import jax.numpy as jnp
from jax.experimental import pallas as pl
from jax.experimental.pallas import tpu as pltpu
from jax.experimental.pallas import tpu_sc as plsc

_N_TOKENS = 16384
_INPUT_DIM = 256
_HIDDEN = 64
_K = 1024
_S = 3
_TN = 512


def _argmin_like_ref(r, cbt, iota):
    f32 = jnp.float32
    cn = jnp.sum(cbt * cbt, axis=0, keepdims=True)
    rn = jnp.sum(r * r, axis=1, keepdims=True)
    ab = jnp.dot(r, cbt, preferred_element_type=f32)
    d = jnp.sqrt(jnp.maximum((rn + cn) - 2.0 * ab, 0.0))
    m = jnp.min(d, axis=1, keepdims=True)
    return jnp.min(jnp.where(d == m, iota, _K), axis=1, keepdims=True)


def _b0_body(x_ref, w1_ref, b1_ref, w2_ref, b2_ref, cbt_ref, h_ref, idx_ref):
    f32 = jnp.float32
    x = x_ref[...]
    h1 = jnp.maximum(jnp.dot(x, w1_ref[...], preferred_element_type=f32)
                     + b1_ref[...], 0.0)
    h = jnp.dot(h1, w2_ref[...], preferred_element_type=f32) + b2_ref[...]
    h_ref[...] = h
    iota = jax.lax.broadcasted_iota(jnp.int32, (_TN, _K), 1)
    idx_ref[...] = _argmin_like_ref(h, cbt_ref[...], iota)


def _bs_body(r_prev_ref, sel_ref, cbt_ref, r_ref, idx_ref):
    r = r_prev_ref[...] - sel_ref[:, :_HIDDEN]
    r_ref[...] = r
    iota = jax.lax.broadcasted_iota(jnp.int32, (_TN, _K), 1)
    idx_ref[...] = _argmin_like_ref(r, cbt_ref[...], iota)


def _d_body(s0_ref, s1_ref, s2_ref, w3_ref, b3_ref, w4_ref, b4_ref,
            q_ref, rec_ref):
    f32 = jnp.float32
    q = (s0_ref[:, :_HIDDEN] + s1_ref[:, :_HIDDEN]) + s2_ref[:, :_HIDDEN]
    q_ref[...] = q
    d1 = jnp.maximum(jnp.dot(q, w3_ref[...], preferred_element_type=f32)
                     + b3_ref[...], 0.0)
    rec_ref[...] = jnp.dot(d1, w4_ref[...], preferred_element_type=f32) \
        + b4_ref[...]


def _full(shape):
    return pl.BlockSpec(shape, lambda i: (0,) * len(shape))


def _tok(cols):
    return pl.BlockSpec((_TN, cols), lambda i: (i, 0))


_GRID = (_N_TOKENS // _TN,)


def _sc_gather(table, idx):
    info = plsc.get_sparse_core_info()
    nw = info.num_cores * info.num_subcores
    bsz, dim = idx.shape[0], table.shape[1]
    bpw = bsz // nw
    mesh = plsc.VectorSubcoreMesh(core_axis_name="c", subcore_axis_name="s")

    @functools.partial(
        pl.kernel, mesh=mesh,
        out_type=jax.ShapeDtypeStruct((bsz, dim), jnp.float32),
        scratch_types=[
            pltpu.VMEM((bpw,), jnp.int32),
            pltpu.VMEM((bpw, dim), jnp.float32),
            pltpu.SemaphoreType.DMA,
        ],
    )
    def k(table_hbm, idx_hbm, out_hbm, idx_v, rows_v, sem):
        wid = jax.lax.axis_index("s") * info.num_cores + jax.lax.axis_index("c")
        base = wid * bpw
        pltpu.sync_copy(idx_hbm.at[pl.ds(base, bpw)], idx_v)
        pltpu.async_copy(table_hbm.at[idx_v], rows_v, sem).wait()
        pltpu.sync_copy(rows_v, out_hbm.at[pl.ds(base, bpw)])

    return k(table, idx)


def kernel(x, W1, b1, W2, b2, W3, b3, W4, b4, codebooks):
    f32 = jnp.float32
    cbt = jnp.swapaxes(codebooks, 1, 2)
    cbp = jnp.pad(codebooks, ((0, 0), (0, 0), (0, 128 - _HIDDEN)))

    h, idx0 = pl.pallas_call(
        _b0_body,
        grid=_GRID,
        in_specs=[_tok(_INPUT_DIM), _full((_INPUT_DIM, 2 * _HIDDEN)),
                  _full((1, 2 * _HIDDEN)), _full((2 * _HIDDEN, _HIDDEN)),
                  _full((1, _HIDDEN)), _full((_HIDDEN, _K))],
        out_specs=[_tok(_HIDDEN), _tok(1)],
        out_shape=[jax.ShapeDtypeStruct((_N_TOKENS, _HIDDEN), f32),
                   jax.ShapeDtypeStruct((_N_TOKENS, 1), jnp.int32)],
    )(x, W1, b1.reshape(1, -1), W2, b2.reshape(1, -1), cbt[0])

    sel0 = _sc_gather(cbp[0], idx0.reshape(-1))

    r1, idx1 = pl.pallas_call(
        _bs_body,
        grid=_GRID,
        in_specs=[_tok(_HIDDEN), _tok(128), _full((_HIDDEN, _K))],
        out_specs=[_tok(_HIDDEN), _tok(1)],
        out_shape=[jax.ShapeDtypeStruct((_N_TOKENS, _HIDDEN), f32),
                   jax.ShapeDtypeStruct((_N_TOKENS, 1), jnp.int32)],
    )(h, sel0, cbt[1])

    sel1 = _sc_gather(cbp[1], idx1.reshape(-1))

    _, idx2 = pl.pallas_call(
        _bs_body,
        grid=_GRID,
        in_specs=[_tok(_HIDDEN), _tok(128), _full((_HIDDEN, _K))],
        out_specs=[_tok(_HIDDEN), _tok(1)],
        out_shape=[jax.ShapeDtypeStruct((_N_TOKENS, _HIDDEN), f32),
                   jax.ShapeDtypeStruct((_N_TOKENS, 1), jnp.int32)],
    )(r1, sel1, cbt[2])

    sel2 = _sc_gather(cbp[2], idx2.reshape(-1))

    q, rec = pl.pallas_call(
        _d_body,
        grid=_GRID,
        in_specs=[_tok(128), _tok(128), _tok(128),
                  _full((_HIDDEN, 2 * _HIDDEN)), _full((1, 2 * _HIDDEN)),
                  _full((2 * _HIDDEN, _INPUT_DIM)), _full((1, _INPUT_DIM))],
        out_specs=[_tok(_HIDDEN), _tok(_INPUT_DIM)],
        out_shape=[jax.ShapeDtypeStruct((_N_TOKENS, _HIDDEN), f32),
                   jax.ShapeDtypeStruct((_N_TOKENS, _INPUT_DIM), f32)],
    )(sel0, sel1, sel2, W3, b3.reshape(1, -1), W4, b4.reshape(1, -1))

    idx = jnp.concatenate([idx0, idx1, idx2], axis=1)
    return (q, idx.T, rec)


# speedup vs baseline: 2.7110x; 2.7110x over previous
import jax
import jax.numpy as jnp
from jax.experimental import pallas as pl

_N_TOKENS = 16384
_INPUT_DIM = 256
_HIDDEN = 64
_K = 1024
_S = 3
_TN = 512


def _rvq_body(x_ref, w1_ref, b1_ref, w2_ref, b2_ref, w3_ref, b3_ref,
              w4_ref, b4_ref, cbt2_ref, hi_ref, mid_ref, lo_ref,
              q_ref, idx_ref, rec_ref):
    f32 = jnp.float32
    x = x_ref[...]
    h1 = jnp.maximum(jnp.dot(x, w1_ref[...], preferred_element_type=f32)
                     + b1_ref[...], 0.0)
    h = jnp.dot(h1, w2_ref[...], preferred_element_type=f32) + b2_ref[...]

    r = h
    q = jnp.zeros_like(h)
    iota = jax.lax.broadcasted_iota(jnp.int32, (_TN, _K), 1)
    for s in range(_S):
        cbt2 = cbt2_ref[s]
        cn = 0.25 * jnp.sum(cbt2 * cbt2, axis=0, keepdims=True)
        rn = jnp.sum(r * r, axis=1, keepdims=True)
        ab2 = jnp.dot(r, cbt2, preferred_element_type=f32)
        d = jnp.sqrt(jnp.maximum((rn + cn) + ab2, 0.0))
        m = jnp.min(d, axis=1, keepdims=True)
        idx = jnp.min(jnp.where(d == m, iota, _K), axis=1, keepdims=True)
        idx_ref[:, s:s + 1] = idx
        oh = (iota == idx).astype(jnp.bfloat16)
        sel = (jnp.dot(oh, hi_ref[s], preferred_element_type=f32)
               + jnp.dot(oh, mid_ref[s], preferred_element_type=f32)
               + jnp.dot(oh, lo_ref[s], preferred_element_type=f32))
        q = q + sel
        r = r - sel

    q_ref[...] = q
    d1 = jnp.maximum(jnp.dot(q, w3_ref[...], preferred_element_type=f32)
                     + b3_ref[...], 0.0)
    rec_ref[...] = jnp.dot(d1, w4_ref[...], preferred_element_type=f32) \
        + b4_ref[...]


def kernel(x, W1, b1, W2, b2, W3, b3, W4, b4, codebooks):
    f32 = jnp.float32
    cbt2 = -2.0 * jnp.swapaxes(codebooks, 1, 2)
    hi = codebooks.astype(jnp.bfloat16)
    rem1 = codebooks - hi.astype(f32)
    mid = rem1.astype(jnp.bfloat16)
    lo = (rem1 - mid.astype(f32)).astype(jnp.bfloat16)

    grid = (_N_TOKENS // _TN,)
    full = lambda shape: pl.BlockSpec(shape, lambda i: (0,) * len(shape))
    q, idx, rec = pl.pallas_call(
        _rvq_body,
        grid=grid,
        in_specs=[
            pl.BlockSpec((_TN, _INPUT_DIM), lambda i: (i, 0)),
            full((_INPUT_DIM, 2 * _HIDDEN)),
            full((1, 2 * _HIDDEN)),
            full((2 * _HIDDEN, _HIDDEN)),
            full((1, _HIDDEN)),
            full((_HIDDEN, 2 * _HIDDEN)),
            full((1, 2 * _HIDDEN)),
            full((2 * _HIDDEN, _INPUT_DIM)),
            full((1, _INPUT_DIM)),
            full((_S, _HIDDEN, _K)),
            full((_S, _K, _HIDDEN)),
            full((_S, _K, _HIDDEN)),
            full((_S, _K, _HIDDEN)),
        ],
        out_specs=[
            pl.BlockSpec((_TN, _HIDDEN), lambda i: (i, 0)),
            pl.BlockSpec((_TN, _S), lambda i: (i, 0)),
            pl.BlockSpec((_TN, _INPUT_DIM), lambda i: (i, 0)),
        ],
        out_shape=[
            jax.ShapeDtypeStruct((_N_TOKENS, _HIDDEN), f32),
            jax.ShapeDtypeStruct((_N_TOKENS, _S), jnp.int32),
            jax.ShapeDtypeStruct((_N_TOKENS, _INPUT_DIM), f32),
        ],
    )(x, W1, b1.reshape(1, -1), W2, b2.reshape(1, -1),
      W3, b3.reshape(1, -1), W4, b4.reshape(1, -1), cbt2, hi, mid, lo)
    return (q, idx.T, rec)
